# baseline (speedup 1.0000x reference)
import functools

import jax
import jax.numpy as jnp
from jax.experimental import pallas as pl
from jax.experimental.pallas import tpu as pltpu


def _spatial_body(x_ref, conf_ref, wenc_ref, bvec_ref, w1_ref, w2_ref,
                  hout_ref, pooled_ref, *, H, W):
    HW = H * W
    hid = wenc_ref.shape[0]

    x = x_ref[0]
    conf = conf_ref[0]

    h = jnp.dot(wenc_ref[...], x, preferred_element_type=jnp.float32)
    h = jnp.maximum(h + bvec_ref[0], 0.0) * conf

    p = jax.lax.broadcasted_iota(jnp.int32, (1, HW), 1)
    xcol = p % W
    yrow = p // W
    m_xm = xcol >= 1
    m_xp = xcol <= W - 2
    m_ym = yrow >= 1
    m_yp = yrow <= H - 2

    def conv3x3(v, w_ref, bias):
        zero = jnp.zeros((), jnp.bfloat16)
        vxm = jnp.where(m_xm, pltpu.roll(v, 1, axis=1), zero)
        vxp = jnp.where(m_xp, pltpu.roll(v, HW - 1, axis=1), zero)
        c3 = jnp.concatenate([vxm, v, vxp], axis=0)
        up = jnp.where(m_ym, pltpu.roll(c3, W, axis=1), zero)
        dn = jnp.where(m_yp, pltpu.roll(c3, HW - W, axis=1), zero)
        u = jnp.concatenate([up, c3, dn], axis=0)
        g = jnp.dot(w_ref[...], u, preferred_element_type=jnp.float32)
        return jnp.maximum(g + bias, 0.0)

    h1 = conv3x3(h.astype(jnp.bfloat16), w1_ref, bvec_ref[1])
    h2 = conv3x3(h1.astype(jnp.bfloat16), w2_ref, bvec_ref[2])

    hout_ref[...] = h2.astype(jnp.bfloat16).reshape(1, hid, HW)

    inv_hw = jnp.full((1, HW), 1.0 / HW, jnp.float32)
    pooled = jax.lax.dot_general(inv_hw, h2, (((1,), (1,)), ((), ())),
                                 preferred_element_type=jnp.float32)
    pooled_ref[...] = pooled.reshape(1, 1, hid)


def _run_spatial(x, conf, wenc_bf, bvec, w1_cat, w2_cat, H, W):
    N, C, HW = x.shape
    hid = wenc_bf.shape[0]
    body = functools.partial(_spatial_body, H=H, W=W)
    return pl.pallas_call(
        body,
        out_shape=(jax.ShapeDtypeStruct((N, hid, HW), jnp.bfloat16),
                   jax.ShapeDtypeStruct((N, 1, hid), jnp.float32)),
        grid=(N,),
        in_specs=[
            pl.BlockSpec((1, C, HW), lambda n: (n, 0, 0)),
            pl.BlockSpec((1, 1, HW), lambda n: (n, 0, 0)),
            pl.BlockSpec((hid, C), lambda n: (0, 0)),
            pl.BlockSpec((3, hid, 1), lambda n: (0, 0, 0)),
            pl.BlockSpec((hid, 9 * hid), lambda n: (0, 0)),
            pl.BlockSpec((hid, 9 * hid), lambda n: (0, 0)),
        ],
        out_specs=(
            pl.BlockSpec((1, hid, HW), lambda n: (n, 0, 0)),
            pl.BlockSpec((1, 1, hid), lambda n: (n, 0, 0)),
        ),
        compiler_params=pltpu.CompilerParams(dimension_semantics=("parallel",)),
    )(x, conf, wenc_bf, bvec, w1_cat, w2_cat)


def _gat_body(pooled_ref, ei_ref, wgat_ref, usrc_ref, udst_ref, bgat_ref,
              wout_ref, c_ref, *, num_layers, heads):
    N = pooled_ref.shape[0]
    hid = bgat_ref.shape[2]
    C = wout_ref.shape[0]
    E = ei_ref.shape[1]
    neg_slope = 0.2

    ii = jax.lax.broadcasted_iota(jnp.int32, (N, E), 0)
    don = (ii == ei_ref[1:2, :]).astype(jnp.float32)
    son = (ii == ei_ref[0:1, :]).astype(jnp.float32)
    cnt = jax.lax.dot_general(don, son, (((1,), (1,)), ((), ())),
                              preferred_element_type=jnp.float32)
    ri = jax.lax.broadcasted_iota(jnp.int32, (N, N), 0)
    ci = jax.lax.broadcasted_iota(jnp.int32, (N, N), 1)
    adj = jnp.logical_or(cnt > 0, ri == ci)

    xg = pooled_ref[...].reshape(N, hid)

    for l in range(num_layers):
        h_all = jnp.dot(xg, wgat_ref[l], preferred_element_type=jnp.float32)
        s_all = jax.lax.dot_general(usrc_ref[l], xg, (((0,), (1,)), ((), ())),
                                    preferred_element_type=jnp.float32)
        d_all = jnp.dot(xg, udst_ref[l], preferred_element_type=jnp.float32)
        acc = jnp.zeros((N, hid), jnp.float32)
        for hd in range(heads):
            e = d_all[:, hd:hd + 1] + s_all[hd:hd + 1, :]
            e = jnp.where(e > 0, e, neg_slope * e)
            e = jnp.where(adj, e, -1e9)
            e = e - jnp.max(e, axis=-1, keepdims=True)
            pr = jnp.exp(e)
            pr = pr / jnp.sum(pr, axis=-1, keepdims=True)
            acc = acc + jnp.dot(pr, h_all[:, hd * hid:(hd + 1) * hid],
                                preferred_element_type=jnp.float32)
        xg = jnp.maximum(acc * (1.0 / heads) + bgat_ref[l], 0.0)

    zt = jax.lax.dot_general(xg, wout_ref[...], (((1,), (1,)), ((), ())),
                             preferred_element_type=jnp.float32)
    c_ref[...] = zt.reshape(N, 1, C)


def _run_gat(pooled, edge_index, w_gat, u_src, u_dst, b_gat, w_out,
             num_layers, heads):
    N = pooled.shape[0]
    hid = pooled.shape[2]
    C = w_out.shape[0]
    E = edge_index.shape[1]
    body = functools.partial(_gat_body, num_layers=num_layers, heads=heads)
    return pl.pallas_call(
        body,
        out_shape=jax.ShapeDtypeStruct((N, 1, C), jnp.float32),
        grid=(1,),
        in_specs=[
            pl.BlockSpec((N, 1, hid), lambda i: (0, 0, 0)),
            pl.BlockSpec((2, E), lambda i: (0, 0)),
            pl.BlockSpec((num_layers, hid, heads * hid), lambda i: (0, 0, 0)),
            pl.BlockSpec((num_layers, hid, heads), lambda i: (0, 0, 0)),
            pl.BlockSpec((num_layers, hid, heads), lambda i: (0, 0, 0)),
            pl.BlockSpec((num_layers, 1, hid), lambda i: (0, 0, 0)),
            pl.BlockSpec((C, hid), lambda i: (0, 0)),
        ],
        out_specs=pl.BlockSpec((N, 1, C), lambda i: (0, 0, 0)),
        compiler_params=pltpu.CompilerParams(dimension_semantics=("arbitrary",)),
    )(pooled, edge_index, w_gat, u_src, u_dst, b_gat, w_out)


def _combine_body(h_ref, c_ref, wout_ref, bout_ref, out_ref, *, H, W):
    C = wout_ref.shape[0]
    HW = h_ref.shape[2]
    y = jnp.dot(wout_ref[...], h_ref[0], preferred_element_type=jnp.float32)
    cn = c_ref[0]
    ones = jnp.full((1, HW), 1.0, jnp.float32)
    corr = jax.lax.dot_general(cn, ones, (((0,), (0,)), ((), ())),
                               preferred_element_type=jnp.float32)
    out_ref[...] = (y + corr + bout_ref[...]).reshape(1, C, H, W)


def _run_combine(hproc, cvec, wout_bf, b_out, H, W):
    N, hid, HW = hproc.shape
    C = wout_bf.shape[0]
    body = functools.partial(_combine_body, H=H, W=W)
    return pl.pallas_call(
        body,
        out_shape=jax.ShapeDtypeStruct((N, C, H, W), jnp.float32),
        grid=(N,),
        in_specs=[
            pl.BlockSpec((1, hid, HW), lambda n: (n, 0, 0)),
            pl.BlockSpec((1, 1, C), lambda n: (n, 0, 0)),
            pl.BlockSpec((C, hid), lambda n: (0, 0)),
            pl.BlockSpec((C, 1), lambda n: (0, 0)),
        ],
        out_specs=pl.BlockSpec((1, C, H, W), lambda n: (n, 0, 0, 0)),
        compiler_params=pltpu.CompilerParams(dimension_semantics=("parallel",)),
    )(hproc, cvec, wout_bf, b_out)


def kernel(x, edge_index, confidence_maps, w_enc, bvec, w_sp1, w_sp2,
           w_gat, u_src, u_dst, b_gat, w_out, b_out):
    N, C, H, W = x.shape
    HW = H * W
    hid = w_enc.shape[0]
    num_layers = w_gat.shape[0]
    heads = u_src.shape[2]

    x_flat = x.reshape(N, C, HW).astype(jnp.bfloat16)
    conf_flat = confidence_maps.reshape(N, 1, HW)

    wenc_bf = w_enc.astype(jnp.bfloat16)
    w1_cat = jnp.transpose(w_sp1.astype(jnp.bfloat16), (1, 0, 2)).reshape(hid, 9 * hid)
    w2_cat = jnp.transpose(w_sp2.astype(jnp.bfloat16), (1, 0, 2)).reshape(hid, 9 * hid)
    wout_bf = w_out.astype(jnp.bfloat16)

    hproc, pooled = _run_spatial(x_flat, conf_flat, wenc_bf, bvec,
                                 w1_cat, w2_cat, H, W)
    cvec = _run_gat(pooled, edge_index, w_gat, u_src, u_dst, b_gat, w_out,
                    num_layers, heads)
    return _run_combine(hproc, cvec, wout_bf, b_out, H, W)


# speedup vs baseline: 1.4148x; 1.4148x over previous
import functools

import jax
import jax.numpy as jnp
from jax.experimental import pallas as pl
from jax.experimental.pallas import tpu as pltpu


def _spatial_body(x_ref, conf_ref, wenc_ref, bvec_ref, w1_ref, w2_ref,
                  hout_ref, pooled_ref, *, H, W):
    HW = H * W
    hid = wenc_ref.shape[0]

    x = x_ref[0]
    conf = conf_ref[0]

    h = jnp.dot(wenc_ref[...], x, preferred_element_type=jnp.float32)
    h = jnp.maximum(h + bvec_ref[0], 0.0) * conf

    p = jax.lax.broadcasted_iota(jnp.int32, (1, HW), 1)
    xcol = p % W
    yrow = p // W
    m_xm = xcol >= 1
    m_xp = xcol <= W - 2
    m_ym = yrow >= 1
    m_yp = yrow <= H - 2

    def conv3x3(v, w_ref, bias):
        zero = jnp.zeros((), jnp.bfloat16)
        vxm = jnp.where(m_xm, pltpu.roll(v, 1, axis=1), zero)
        vxp = jnp.where(m_xp, pltpu.roll(v, HW - 1, axis=1), zero)
        c3 = jnp.concatenate([vxm, v, vxp], axis=0)
        up = jnp.where(m_ym, pltpu.roll(c3, W, axis=1), zero)
        dn = jnp.where(m_yp, pltpu.roll(c3, HW - W, axis=1), zero)
        u = jnp.concatenate([up, c3, dn], axis=0)
        g = jnp.dot(w_ref[...], u, preferred_element_type=jnp.float32)
        return jnp.maximum(g + bias, 0.0)

    h1 = conv3x3(h.astype(jnp.bfloat16), w1_ref, bvec_ref[1])
    h2 = conv3x3(h1.astype(jnp.bfloat16), w2_ref, bvec_ref[2])

    hout_ref[...] = h2.astype(jnp.bfloat16).reshape(1, hid, HW)

    inv_hw = jnp.full((1, HW), 1.0 / HW, jnp.float32)
    pooled = jax.lax.dot_general(inv_hw, h2, (((1,), (1,)), ((), ())),
                                 preferred_element_type=jnp.float32)
    pooled_ref[...] = pooled.reshape(1, 1, hid)


def _run_spatial(x, conf, wenc_bf, bvec, w1_cat, w2_cat, H, W):
    N, C, HW = x.shape
    hid = wenc_bf.shape[0]
    body = functools.partial(_spatial_body, H=H, W=W)
    return pl.pallas_call(
        body,
        out_shape=(jax.ShapeDtypeStruct((N, hid, HW), jnp.bfloat16),
                   jax.ShapeDtypeStruct((N, 1, hid), jnp.float32)),
        grid=(N,),
        in_specs=[
            pl.BlockSpec((1, C, HW), lambda n: (n, 0, 0)),
            pl.BlockSpec((1, 1, HW), lambda n: (n, 0, 0)),
            pl.BlockSpec((hid, C), lambda n: (0, 0)),
            pl.BlockSpec((3, hid, 1), lambda n: (0, 0, 0)),
            pl.BlockSpec((hid, 9 * hid), lambda n: (0, 0)),
            pl.BlockSpec((hid, 9 * hid), lambda n: (0, 0)),
        ],
        out_specs=(
            pl.BlockSpec((1, hid, HW), lambda n: (n, 0, 0)),
            pl.BlockSpec((1, 1, hid), lambda n: (n, 0, 0)),
        ),
        compiler_params=pltpu.CompilerParams(dimension_semantics=("parallel",)),
    )(x, conf, wenc_bf, bvec, w1_cat, w2_cat)


def _gat_body(pooled_ref, ei_ref, wgat_ref, usrc_ref, udst_ref, bgat_ref,
              wout_ref, c_ref, *, num_layers, heads):
    N = pooled_ref.shape[0]
    hid = bgat_ref.shape[2]
    C = wout_ref.shape[0]
    E = ei_ref.shape[1]
    neg_slope = 0.2

    ii = jax.lax.broadcasted_iota(jnp.int32, (N, E), 0)
    don = (ii == ei_ref[1:2, :]).astype(jnp.float32)
    son = (ii == ei_ref[0:1, :]).astype(jnp.float32)
    cnt = jax.lax.dot_general(don, son, (((1,), (1,)), ((), ())),
                              preferred_element_type=jnp.float32)
    ri = jax.lax.broadcasted_iota(jnp.int32, (N, N), 0)
    ci = jax.lax.broadcasted_iota(jnp.int32, (N, N), 1)
    adj = jnp.logical_or(cnt > 0, ri == ci)

    xg = pooled_ref[...].reshape(N, hid)

    for l in range(num_layers):
        h_all = jnp.dot(xg, wgat_ref[l], preferred_element_type=jnp.float32)
        s_all = jax.lax.dot_general(usrc_ref[l], xg, (((0,), (1,)), ((), ())),
                                    preferred_element_type=jnp.float32)
        d_all = jnp.dot(xg, udst_ref[l], preferred_element_type=jnp.float32)
        acc = jnp.zeros((N, hid), jnp.float32)
        for hd in range(heads):
            e = d_all[:, hd:hd + 1] + s_all[hd:hd + 1, :]
            e = jnp.where(e > 0, e, neg_slope * e)
            e = jnp.where(adj, e, -1e9)
            e = e - jnp.max(e, axis=-1, keepdims=True)
            pr = jnp.exp(e)
            pr = pr / jnp.sum(pr, axis=-1, keepdims=True)
            acc = acc + jnp.dot(pr, h_all[:, hd * hid:(hd + 1) * hid],
                                preferred_element_type=jnp.float32)
        xg = jnp.maximum(acc * (1.0 / heads) + bgat_ref[l], 0.0)

    zt = jax.lax.dot_general(xg, wout_ref[...], (((1,), (1,)), ((), ())),
                             preferred_element_type=jnp.float32)
    c_ref[...] = zt.reshape(N, 1, C)


def _run_gat(pooled, edge_index, w_gat, u_src, u_dst, b_gat, w_out,
             num_layers, heads):
    N = pooled.shape[0]
    hid = pooled.shape[2]
    C = w_out.shape[0]
    E = edge_index.shape[1]
    body = functools.partial(_gat_body, num_layers=num_layers, heads=heads)
    return pl.pallas_call(
        body,
        out_shape=jax.ShapeDtypeStruct((N, 1, C), jnp.float32),
        grid=(1,),
        in_specs=[
            pl.BlockSpec((N, 1, hid), lambda i: (0, 0, 0)),
            pl.BlockSpec((2, E), lambda i: (0, 0)),
            pl.BlockSpec((num_layers, hid, heads * hid), lambda i: (0, 0, 0)),
            pl.BlockSpec((num_layers, hid, heads), lambda i: (0, 0, 0)),
            pl.BlockSpec((num_layers, hid, heads), lambda i: (0, 0, 0)),
            pl.BlockSpec((num_layers, 1, hid), lambda i: (0, 0, 0)),
            pl.BlockSpec((C, hid), lambda i: (0, 0)),
        ],
        out_specs=pl.BlockSpec((N, 1, C), lambda i: (0, 0, 0)),
        compiler_params=pltpu.CompilerParams(dimension_semantics=("arbitrary",)),
    )(pooled, edge_index, w_gat, u_src, u_dst, b_gat, w_out)


def _combine_body(h_ref, c_ref, wout_ref, bout_ref, out_ref, *, H, W):
    C = wout_ref.shape[0]
    HW = h_ref.shape[2]
    y = jnp.dot(wout_ref[...], h_ref[0], preferred_element_type=jnp.float32)
    cn = c_ref[0]
    ones = jnp.full((1, HW), 1.0, jnp.float32)
    corr = jax.lax.dot_general(cn, ones, (((0,), (0,)), ((), ())),
                               preferred_element_type=jnp.float32)
    out_ref[...] = (y + corr + bout_ref[...]).reshape(1, C, HW)


def _run_combine(hproc, cvec, wout_bf, b_out, H, W):
    N, hid, HW = hproc.shape
    C = wout_bf.shape[0]
    body = functools.partial(_combine_body, H=H, W=W)
    return pl.pallas_call(
        body,
        out_shape=jax.ShapeDtypeStruct((N, C, HW), jnp.float32),
        grid=(N,),
        in_specs=[
            pl.BlockSpec((1, hid, HW), lambda n: (n, 0, 0)),
            pl.BlockSpec((1, 1, C), lambda n: (n, 0, 0)),
            pl.BlockSpec((C, hid), lambda n: (0, 0)),
            pl.BlockSpec((C, 1), lambda n: (0, 0)),
        ],
        out_specs=pl.BlockSpec((1, C, HW), lambda n: (n, 0, 0)),
        compiler_params=pltpu.CompilerParams(dimension_semantics=("parallel",)),
    )(hproc, cvec, wout_bf, b_out)


def kernel(x, edge_index, confidence_maps, w_enc, bvec, w_sp1, w_sp2,
           w_gat, u_src, u_dst, b_gat, w_out, b_out):
    N, C, H, W = x.shape
    HW = H * W
    hid = w_enc.shape[0]
    num_layers = w_gat.shape[0]
    heads = u_src.shape[2]

    x_flat = x.reshape(N, C, HW).astype(jnp.bfloat16)
    conf_flat = confidence_maps.reshape(N, 1, HW)

    wenc_bf = w_enc.astype(jnp.bfloat16)
    w1_cat = jnp.transpose(w_sp1.astype(jnp.bfloat16), (1, 0, 2)).reshape(hid, 9 * hid)
    w2_cat = jnp.transpose(w_sp2.astype(jnp.bfloat16), (1, 0, 2)).reshape(hid, 9 * hid)
    wout_bf = w_out.astype(jnp.bfloat16)

    hproc, pooled = _run_spatial(x_flat, conf_flat, wenc_bf, bvec,
                                 w1_cat, w2_cat, H, W)
    cvec = _run_gat(pooled, edge_index, w_gat, u_src, u_dst, b_gat, w_out,
                    num_layers, heads)
    out = _run_combine(hproc, cvec, wout_bf, b_out, H, W)
    return out.reshape(N, C, H, W)


# speedup vs baseline: 1.4996x; 1.0599x over previous
import functools

import jax
import jax.numpy as jnp
from jax.experimental import pallas as pl
from jax.experimental.pallas import tpu as pltpu


def _spatial_body(x_ref, conf_ref, wenc_ref, bvec_ref, w1_ref, w2_ref,
                  hout_ref, pooled_ref, *, H, W, P):
    HW = H * W
    L = P * HW
    hid = wenc_ref.shape[0]

    x = x_ref[0]
    conf = conf_ref[0]

    h = jnp.dot(wenc_ref[...], x, preferred_element_type=jnp.float32)
    h = jnp.maximum(h + bvec_ref[0], 0.0) * conf

    p = jax.lax.broadcasted_iota(jnp.int32, (1, L), 1)
    xcol = p % W
    yrow = (p // W) % H
    m_xm = xcol >= 1
    m_xp = xcol <= W - 2
    m_ym = yrow >= 1
    m_yp = yrow <= H - 2

    def conv3x3(v, w_ref, bias):
        zero = jnp.zeros((), jnp.bfloat16)
        vxm = jnp.where(m_xm, pltpu.roll(v, 1, axis=1), zero)
        vxp = jnp.where(m_xp, pltpu.roll(v, L - 1, axis=1), zero)
        c3 = jnp.concatenate([vxm, v, vxp], axis=0)
        K3 = 3 * v.shape[0]
        g_m1 = jnp.dot(w_ref[:, :K3], c3, preferred_element_type=jnp.float32)
        g_0 = jnp.dot(w_ref[:, K3:2 * K3], c3, preferred_element_type=jnp.float32)
        g_p1 = jnp.dot(w_ref[:, 2 * K3:], c3, preferred_element_type=jnp.float32)
        zf = jnp.zeros((), jnp.float32)
        g = g_0 + jnp.where(m_ym, pltpu.roll(g_m1, W, axis=1), zf)
        g = g + jnp.where(m_yp, pltpu.roll(g_p1, L - W, axis=1), zf)
        return jnp.maximum(g + bias, 0.0)

    h1 = conv3x3(h.astype(jnp.bfloat16), w1_ref, bvec_ref[1])
    h2 = conv3x3(h1.astype(jnp.bfloat16), w2_ref, bvec_ref[2])

    hout_ref[...] = h2.astype(jnp.bfloat16).reshape(1, hid, L)

    sel = (jax.lax.broadcasted_iota(jnp.int32, (P, L), 1) // HW ==
           jax.lax.broadcasted_iota(jnp.int32, (P, L), 0))
    selw = jnp.where(sel, 1.0 / HW, 0.0)
    pooled = jax.lax.dot_general(selw, h2, (((1,), (1,)), ((), ())),
                                 preferred_element_type=jnp.float32)
    pooled_ref[...] = pooled.reshape(P, 1, hid)


def _run_spatial(x, conf, wenc_bf, bvec, w1_cat, w2_cat, H, W, P):
    M, C, L = x.shape
    N = M * P
    hid = wenc_bf.shape[0]
    body = functools.partial(_spatial_body, H=H, W=W, P=P)
    return pl.pallas_call(
        body,
        out_shape=(jax.ShapeDtypeStruct((M, hid, L), jnp.bfloat16),
                   jax.ShapeDtypeStruct((N, 1, hid), jnp.float32)),
        grid=(M,),
        in_specs=[
            pl.BlockSpec((1, C, L), lambda n: (n, 0, 0)),
            pl.BlockSpec((1, 1, L), lambda n: (n, 0, 0)),
            pl.BlockSpec((hid, C), lambda n: (0, 0)),
            pl.BlockSpec((3, hid, 1), lambda n: (0, 0, 0)),
            pl.BlockSpec((hid, 9 * hid), lambda n: (0, 0)),
            pl.BlockSpec((hid, 9 * hid), lambda n: (0, 0)),
        ],
        out_specs=(
            pl.BlockSpec((1, hid, L), lambda n: (n, 0, 0)),
            pl.BlockSpec((P, 1, hid), lambda n: (n, 0, 0)),
        ),
        compiler_params=pltpu.CompilerParams(dimension_semantics=("parallel",)),
    )(x, conf, wenc_bf, bvec, w1_cat, w2_cat)


def _gat_body(pooled_ref, ei_ref, wgat_ref, usrc_ref, udst_ref, bgat_ref,
              wout_ref, c_ref, *, num_layers, heads, P):
    N = pooled_ref.shape[0]
    hid = bgat_ref.shape[2]
    C = wout_ref.shape[0]
    E = ei_ref.shape[1]
    neg_slope = 0.2

    ii = jax.lax.broadcasted_iota(jnp.int32, (N, E), 0)
    don = (ii == ei_ref[1:2, :]).astype(jnp.float32)
    son = (ii == ei_ref[0:1, :]).astype(jnp.float32)
    cnt = jax.lax.dot_general(don, son, (((1,), (1,)), ((), ())),
                              preferred_element_type=jnp.float32)
    ri = jax.lax.broadcasted_iota(jnp.int32, (N, N), 0)
    ci = jax.lax.broadcasted_iota(jnp.int32, (N, N), 1)
    adj = jnp.logical_or(cnt > 0, ri == ci)

    xg = pooled_ref[...].reshape(N, hid)

    for l in range(num_layers):
        h_all = jnp.dot(xg, wgat_ref[l], preferred_element_type=jnp.float32)
        s_all = jax.lax.dot_general(usrc_ref[l], xg, (((0,), (1,)), ((), ())),
                                    preferred_element_type=jnp.float32)
        d_all = jnp.dot(xg, udst_ref[l], preferred_element_type=jnp.float32)
        acc = jnp.zeros((N, hid), jnp.float32)
        for hd in range(heads):
            e = d_all[:, hd:hd + 1] + s_all[hd:hd + 1, :]
            e = jnp.where(e > 0, e, neg_slope * e)
            e = jnp.where(adj, e, -1e9)
            e = e - jnp.max(e, axis=-1, keepdims=True)
            pr = jnp.exp(e)
            pr = pr / jnp.sum(pr, axis=-1, keepdims=True)
            acc = acc + jnp.dot(pr, h_all[:, hd * hid:(hd + 1) * hid],
                                preferred_element_type=jnp.float32)
        xg = jnp.maximum(acc * (1.0 / heads) + bgat_ref[l], 0.0)

    zt = jax.lax.dot_general(xg, wout_ref[...], (((1,), (1,)), ((), ())),
                             preferred_element_type=jnp.float32)
    c_ref[...] = zt.reshape(N // P, P, C)


def _run_gat(pooled, edge_index, w_gat, u_src, u_dst, b_gat, w_out,
             num_layers, heads, P):
    N = pooled.shape[0]
    hid = pooled.shape[2]
    C = w_out.shape[0]
    E = edge_index.shape[1]
    body = functools.partial(_gat_body, num_layers=num_layers, heads=heads, P=P)
    return pl.pallas_call(
        body,
        out_shape=jax.ShapeDtypeStruct((N // P, P, C), jnp.float32),
        grid=(1,),
        in_specs=[
            pl.BlockSpec((N, 1, hid), lambda i: (0, 0, 0)),
            pl.BlockSpec((2, E), lambda i: (0, 0)),
            pl.BlockSpec((num_layers, hid, heads * hid), lambda i: (0, 0, 0)),
            pl.BlockSpec((num_layers, hid, heads), lambda i: (0, 0, 0)),
            pl.BlockSpec((num_layers, hid, heads), lambda i: (0, 0, 0)),
            pl.BlockSpec((num_layers, 1, hid), lambda i: (0, 0, 0)),
            pl.BlockSpec((C, hid), lambda i: (0, 0)),
        ],
        out_specs=pl.BlockSpec((N // P, P, C), lambda i: (0, 0, 0)),
        compiler_params=pltpu.CompilerParams(dimension_semantics=("arbitrary",)),
    )(pooled, edge_index, w_gat, u_src, u_dst, b_gat, w_out)


def _combine_body(h_ref, c_ref, wout_ref, bout_ref, out_ref, *, HW, P):
    C = wout_ref.shape[0]
    L = h_ref.shape[2]
    y = jnp.dot(wout_ref[...], h_ref[0], preferred_element_type=jnp.float32)
    cn = c_ref[0]
    sel = (jax.lax.broadcasted_iota(jnp.int32, (P, L), 1) // HW ==
           jax.lax.broadcasted_iota(jnp.int32, (P, L), 0))
    selw = jnp.where(sel, 1.0, 0.0)
    corr = jax.lax.dot_general(cn, selw, (((0,), (0,)), ((), ())),
                               preferred_element_type=jnp.float32)
    out_ref[...] = (y + corr + bout_ref[...]).reshape(1, C, L)


def _run_combine(hproc, cvec, wout_bf, b_out, HW, P):
    M, hid, L = hproc.shape
    C = wout_bf.shape[0]
    body = functools.partial(_combine_body, HW=HW, P=P)
    return pl.pallas_call(
        body,
        out_shape=jax.ShapeDtypeStruct((M, C, L), jnp.float32),
        grid=(M,),
        in_specs=[
            pl.BlockSpec((1, hid, L), lambda n: (n, 0, 0)),
            pl.BlockSpec((1, P, C), lambda n: (n, 0, 0)),
            pl.BlockSpec((C, hid), lambda n: (0, 0)),
            pl.BlockSpec((C, 1), lambda n: (0, 0)),
        ],
        out_specs=pl.BlockSpec((1, C, L), lambda n: (n, 0, 0)),
        compiler_params=pltpu.CompilerParams(dimension_semantics=("parallel",)),
    )(hproc, cvec, wout_bf, b_out)


def kernel(x, edge_index, confidence_maps, w_enc, bvec, w_sp1, w_sp2,
           w_gat, u_src, u_dst, b_gat, w_out, b_out):
    N, C, H, W = x.shape
    HW = H * W
    hid = w_enc.shape[0]
    num_layers = w_gat.shape[0]
    heads = u_src.shape[2]
    P = 2 if N % 2 == 0 else 1
    M = N // P

    x_flat = jnp.transpose(x.reshape(M, P, C, HW), (0, 2, 1, 3)) \
        .reshape(M, C, P * HW).astype(jnp.bfloat16)
    conf_flat = jnp.transpose(confidence_maps.reshape(M, P, 1, HW),
                              (0, 2, 1, 3)).reshape(M, 1, P * HW)

    wenc_bf = w_enc.astype(jnp.bfloat16)
    w1_cat = jnp.transpose(w_sp1.astype(jnp.bfloat16), (1, 0, 2)).reshape(hid, 9 * hid)
    w2_cat = jnp.transpose(w_sp2.astype(jnp.bfloat16), (1, 0, 2)).reshape(hid, 9 * hid)
    wout_bf = w_out.astype(jnp.bfloat16)

    hproc, pooled = _run_spatial(x_flat, conf_flat, wenc_bf, bvec,
                                 w1_cat, w2_cat, H, W, P)
    cvec = _run_gat(pooled, edge_index, w_gat, u_src, u_dst, b_gat, w_out,
                    num_layers, heads, P)
    out = _run_combine(hproc, cvec, wout_bf, b_out, HW, P)
    return jnp.transpose(out.reshape(M, C, P, H, W), (0, 2, 1, 3, 4)) \
        .reshape(N, C, H, W)


# speedup vs baseline: 1.6670x; 1.1116x over previous
import functools

import jax
import jax.numpy as jnp
from jax.experimental import pallas as pl
from jax.experimental.pallas import tpu as pltpu


def _spatial_body(x_ref, conf_ref, wenc_ref, bvec_ref, w1_ref, w2_ref,
                  hout_ref, pooled_ref, *, H, W, P):
    HW = H * W
    L = P * HW
    hid = wenc_ref.shape[0]

    x = x_ref[0]
    conf = conf_ref[0]

    h = jnp.dot(wenc_ref[...], x, preferred_element_type=jnp.float32)
    h = jnp.maximum(h + bvec_ref[0], 0.0) * conf

    p = jax.lax.broadcasted_iota(jnp.int32, (1, L), 1)
    xcol = p % W
    yrow = (p // W) % H
    m_xm = xcol >= 1
    m_xp = xcol <= W - 2
    m_ym = yrow >= 1
    m_yp = yrow <= H - 2

    def conv3x3(v, w_ref, bias):
        zero = jnp.zeros((), jnp.bfloat16)
        vxm = jnp.where(m_xm, pltpu.roll(v, 1, axis=1), zero)
        vxp = jnp.where(m_xp, pltpu.roll(v, L - 1, axis=1), zero)
        c3 = jnp.concatenate([vxm, v, vxp], axis=0)
        K3 = 3 * v.shape[0]
        g_m1 = jnp.dot(w_ref[:, :K3], c3, preferred_element_type=jnp.float32)
        g_0 = jnp.dot(w_ref[:, K3:2 * K3], c3, preferred_element_type=jnp.float32)
        g_p1 = jnp.dot(w_ref[:, 2 * K3:], c3, preferred_element_type=jnp.float32)
        zf = jnp.zeros((), jnp.float32)
        g = g_0 + jnp.where(m_ym, pltpu.roll(g_m1, W, axis=1), zf)
        g = g + jnp.where(m_yp, pltpu.roll(g_p1, L - W, axis=1), zf)
        return jnp.maximum(g + bias, 0.0)

    h1 = conv3x3(h.astype(jnp.bfloat16), w1_ref, bvec_ref[1])
    h2 = conv3x3(h1.astype(jnp.bfloat16), w2_ref, bvec_ref[2])

    hout_ref[...] = h2.astype(jnp.bfloat16).reshape(1, hid, L)

    sel = (jax.lax.broadcasted_iota(jnp.int32, (P, L), 1) // HW ==
           jax.lax.broadcasted_iota(jnp.int32, (P, L), 0))
    selw = jnp.where(sel, 1.0 / HW, 0.0)
    pooled = jax.lax.dot_general(selw, h2, (((1,), (1,)), ((), ())),
                                 preferred_element_type=jnp.float32)
    pooled_ref[...] = pooled.reshape(P, 1, hid)


def _run_spatial(x, conf, wenc_bf, bvec, w1_cat, w2_cat, H, W, P):
    M, C, L = x.shape
    N = M * P
    hid = wenc_bf.shape[0]
    body = functools.partial(_spatial_body, H=H, W=W, P=P)
    return pl.pallas_call(
        body,
        out_shape=(jax.ShapeDtypeStruct((M, hid, L), jnp.bfloat16),
                   jax.ShapeDtypeStruct((N, 1, hid), jnp.float32)),
        grid=(M,),
        in_specs=[
            pl.BlockSpec((1, C, L), lambda n: (n, 0, 0)),
            pl.BlockSpec((1, 1, L), lambda n: (n, 0, 0)),
            pl.BlockSpec((hid, C), lambda n: (0, 0)),
            pl.BlockSpec((3, hid, 1), lambda n: (0, 0, 0)),
            pl.BlockSpec((hid, 9 * hid), lambda n: (0, 0)),
            pl.BlockSpec((hid, 9 * hid), lambda n: (0, 0)),
        ],
        out_specs=(
            pl.BlockSpec((1, hid, L), lambda n: (n, 0, 0)),
            pl.BlockSpec((P, 1, hid), lambda n: (n, 0, 0)),
        ),
        compiler_params=pltpu.CompilerParams(dimension_semantics=("parallel",)),
    )(x, conf, wenc_bf, bvec, w1_cat, w2_cat)


def _gat_body(pooled_ref, ei_ref, wgat_ref, usrc_ref, udst_ref, bgat_ref,
              wout_ref, c_ref, *, num_layers, heads, P):
    N = pooled_ref.shape[0]
    hid = bgat_ref.shape[2]
    C = wout_ref.shape[0]
    E = ei_ref.shape[1]
    neg_slope = 0.2

    ii = jax.lax.broadcasted_iota(jnp.int32, (N, E), 0)
    don = (ii == ei_ref[1:2, :]).astype(jnp.float32)
    son = (ii == ei_ref[0:1, :]).astype(jnp.float32)
    cnt = jax.lax.dot_general(don, son, (((1,), (1,)), ((), ())),
                              preferred_element_type=jnp.float32)
    ri = jax.lax.broadcasted_iota(jnp.int32, (N, N), 0)
    ci = jax.lax.broadcasted_iota(jnp.int32, (N, N), 1)
    adj = jnp.logical_or(cnt > 0, ri == ci)

    xg = pooled_ref[...].reshape(N, hid)

    for l in range(num_layers):
        h_all = jnp.dot(xg, wgat_ref[l], preferred_element_type=jnp.float32)
        s_all = jax.lax.dot_general(usrc_ref[l], xg, (((0,), (1,)), ((), ())),
                                    preferred_element_type=jnp.float32)
        d_all = jnp.dot(xg, udst_ref[l], preferred_element_type=jnp.float32)
        acc = jnp.zeros((N, hid), jnp.float32)
        for hd in range(heads):
            e = d_all[:, hd:hd + 1] + s_all[hd:hd + 1, :]
            e = jnp.where(e > 0, e, neg_slope * e)
            e = jnp.where(adj, e, -1e9)
            e = e - jnp.max(e, axis=-1, keepdims=True)
            pr = jnp.exp(e)
            pr = pr / jnp.sum(pr, axis=-1, keepdims=True)
            acc = acc + jnp.dot(pr, h_all[:, hd * hid:(hd + 1) * hid],
                                preferred_element_type=jnp.float32)
        xg = jnp.maximum(acc * (1.0 / heads) + bgat_ref[l], 0.0)

    zt = jax.lax.dot_general(xg, wout_ref[...], (((1,), (1,)), ((), ())),
                             preferred_element_type=jnp.float32)
    c_ref[...] = zt.reshape(N // P, P, C)


def _run_gat(pooled, edge_index, w_gat, u_src, u_dst, b_gat, w_out,
             num_layers, heads, P):
    N = pooled.shape[0]
    hid = pooled.shape[2]
    C = w_out.shape[0]
    E = edge_index.shape[1]
    body = functools.partial(_gat_body, num_layers=num_layers, heads=heads, P=P)
    return pl.pallas_call(
        body,
        out_shape=jax.ShapeDtypeStruct((N // P, P, C), jnp.float32),
        grid=(1,),
        in_specs=[
            pl.BlockSpec((N, 1, hid), lambda i: (0, 0, 0)),
            pl.BlockSpec((2, E), lambda i: (0, 0)),
            pl.BlockSpec((num_layers, hid, heads * hid), lambda i: (0, 0, 0)),
            pl.BlockSpec((num_layers, hid, heads), lambda i: (0, 0, 0)),
            pl.BlockSpec((num_layers, hid, heads), lambda i: (0, 0, 0)),
            pl.BlockSpec((num_layers, 1, hid), lambda i: (0, 0, 0)),
            pl.BlockSpec((C, hid), lambda i: (0, 0)),
        ],
        out_specs=pl.BlockSpec((N // P, P, C), lambda i: (0, 0, 0)),
        compiler_params=pltpu.CompilerParams(dimension_semantics=("arbitrary",)),
    )(pooled, edge_index, w_gat, u_src, u_dst, b_gat, w_out)


def _combine_body(h_ref, c_ref, wout_ref, bout_ref, out_ref, *, HW, P, SB):
    C = wout_ref.shape[0]
    L = h_ref.shape[2]
    sel = (jax.lax.broadcasted_iota(jnp.int32, (P, L), 1) // HW ==
           jax.lax.broadcasted_iota(jnp.int32, (P, L), 0))
    selw = jnp.where(sel, 1.0, 0.0)
    for k in range(SB):
        y = jnp.dot(wout_ref[...], h_ref[k], preferred_element_type=jnp.float32)
        cn = c_ref[k]
        corr = jax.lax.dot_general(cn, selw, (((0,), (0,)), ((), ())),
                                   preferred_element_type=jnp.float32)
        out_ref[k] = y + corr + bout_ref[...]


def _run_combine(hproc, cvec, wout_bf, b_out, HW, P, SB):
    M, hid, L = hproc.shape
    C = wout_bf.shape[0]
    body = functools.partial(_combine_body, HW=HW, P=P, SB=SB)
    return pl.pallas_call(
        body,
        out_shape=jax.ShapeDtypeStruct((M, C, L), jnp.float32),
        grid=(M // SB,),
        in_specs=[
            pl.BlockSpec((SB, hid, L), lambda n: (n, 0, 0)),
            pl.BlockSpec((SB, P, C), lambda n: (n, 0, 0)),
            pl.BlockSpec((C, hid), lambda n: (0, 0)),
            pl.BlockSpec((C, 1), lambda n: (0, 0)),
        ],
        out_specs=pl.BlockSpec((SB, C, L), lambda n: (n, 0, 0)),
        compiler_params=pltpu.CompilerParams(dimension_semantics=("parallel",)),
    )(hproc, cvec, wout_bf, b_out)


def kernel(x, edge_index, confidence_maps, w_enc, bvec, w_sp1, w_sp2,
           w_gat, u_src, u_dst, b_gat, w_out, b_out):
    N, C, H, W = x.shape
    HW = H * W
    hid = w_enc.shape[0]
    num_layers = w_gat.shape[0]
    heads = u_src.shape[2]
    P = 4 if N % 4 == 0 else 1
    M = N // P
    SB = 2 if M % 2 == 0 else 1

    x_flat = jnp.transpose(x.reshape(M, P, C, HW), (0, 2, 1, 3)) \
        .reshape(M, C, P * HW).astype(jnp.bfloat16)
    conf_flat = jnp.transpose(confidence_maps.reshape(M, P, 1, HW),
                              (0, 2, 1, 3)).reshape(M, 1, P * HW)

    wenc_bf = w_enc.astype(jnp.bfloat16)
    w1_cat = jnp.transpose(w_sp1.astype(jnp.bfloat16), (1, 0, 2)).reshape(hid, 9 * hid)
    w2_cat = jnp.transpose(w_sp2.astype(jnp.bfloat16), (1, 0, 2)).reshape(hid, 9 * hid)
    wout_bf = w_out.astype(jnp.bfloat16)

    hproc, pooled = _run_spatial(x_flat, conf_flat, wenc_bf, bvec,
                                 w1_cat, w2_cat, H, W, P)
    cvec = _run_gat(pooled, edge_index, w_gat, u_src, u_dst, b_gat, w_out,
                    num_layers, heads, P)
    out = _run_combine(hproc, cvec, wout_bf, b_out, HW, P, SB)
    return jnp.transpose(out.reshape(M, C, P, H, W), (0, 2, 1, 3, 4)) \
        .reshape(N, C, H, W)
